# baseline (speedup 1.0000x reference)
import functools

import jax
import jax.numpy as jnp
from jax import lax
from jax.experimental import pallas as pl
from jax.experimental.pallas import tpu as pltpu
from jax.experimental.pallas import tpu_sc as plsc

B, C, H, W = 16, 256, 32, 32
HW = H * W
N = B * HW
K = 1024
D = C

NC = 2
NS = 16
NW = NC * NS
ROWS_PER_W = N // NW
CHUNK = 128
NCHUNK = ROWS_PER_W // CHUNK


def _tc_body(emb_ref, z_ref, idx_ref, stats_ref, counts_ref, acc_ref):
    b = pl.program_id(0)
    emb = emb_ref[...]
    zb = z_ref[0]
    s_t = jax.lax.dot_general(
        emb, zb, (((1,), (0,)), ((), ())),
        preferred_element_type=jnp.float32,
        precision=lax.Precision.DEFAULT,
    )
    enorm = jnp.sum(emb * emb, axis=1, keepdims=True)
    znorm = jnp.sum(zb * zb, axis=0, keepdims=True)
    dist_t = (znorm - 2.0 * s_t) + enorm
    m = jnp.min(dist_t, axis=0, keepdims=True)
    ks = lax.broadcasted_iota(jnp.int32, (K, HW), 0)
    idx = jnp.min(jnp.where(dist_t == m, ks, K), axis=0).astype(jnp.int32)
    idx_ref[0, 0, :] = idx
    hits = (ks == idx[None, :]).astype(jnp.float32)

    @pl.when(b == 0)
    def _init():
        counts_ref[...] = jnp.zeros_like(counts_ref)
        acc_ref[0] = 0.0

    counts_ref[...] += jnp.sum(hits, axis=1, keepdims=True)
    acc_ref[0] += jnp.sum(dist_t)

    @pl.when(b == B - 1)
    def _finalize():
        e_mean = counts_ref[...] * (1.0 / N)
        ent = jnp.sum(e_mean * jnp.log(e_mean + 1e-10))
        stats_ref[0] = jnp.exp(-ent)
        stats_ref[1] = acc_ref[0] * (1.0 / (N * K))


_tc_call = pl.pallas_call(
    _tc_body,
    grid=(B,),
    in_specs=[
        pl.BlockSpec((K, D), lambda b: (0, 0)),
        pl.BlockSpec((1, C, HW), lambda b: (b, 0, 0)),
    ],
    out_specs=[
        pl.BlockSpec((1, 1, HW), lambda b: (b, 0, 0)),
        pl.BlockSpec(memory_space=pltpu.SMEM),
    ],
    out_shape=[
        jax.ShapeDtypeStruct((B, 1, HW), jnp.int32),
        jax.ShapeDtypeStruct((2,), jnp.float32),
    ],
    scratch_shapes=[
        pltpu.VMEM((K, 1), jnp.float32),
        pltpu.SMEM((2,), jnp.float32),
    ],
)


def _sc_gather_body(emb_hbm, idx_hbm, out_hbm, idx_v, buf0, buf1, sem0, sem1):
    c = lax.axis_index("c")
    s = lax.axis_index("s")
    wid = s * NC + c
    base = wid * ROWS_PER_W
    pltpu.sync_copy(idx_hbm.at[pl.ds(base, ROWS_PER_W)], idx_v)
    bufs = (buf0, buf1)
    sems = (sem0, sem1)
    copies = [None, None]
    copies[0] = pltpu.async_copy(
        emb_hbm.at[idx_v.at[pl.ds(0, CHUNK)]], bufs[0], sems[0])
    for j in range(NCHUNK):
        cur = j % 2
        if j + 1 < NCHUNK:
            nxt = (j + 1) % 2
            copies[nxt] = pltpu.async_copy(
                emb_hbm.at[idx_v.at[pl.ds((j + 1) * CHUNK, CHUNK)]],
                bufs[nxt], sems[nxt])
        copies[cur].wait()
        pltpu.sync_copy(bufs[cur], out_hbm.at[pl.ds(base + j * CHUNK, CHUNK)])


@functools.lru_cache(maxsize=1)
def _make_sc_gather():
    return pl.kernel(
        _sc_gather_body,
        out_type=jax.ShapeDtypeStruct((N, D), jnp.float32),
        mesh=plsc.VectorSubcoreMesh(
            core_axis_name="c", subcore_axis_name="s",
            num_cores=NC, num_subcores=NS),
        scratch_types=[
            pltpu.VMEM((ROWS_PER_W,), jnp.int32),
            pltpu.VMEM((CHUNK, D), jnp.float32),
            pltpu.VMEM((CHUNK, D), jnp.float32),
            pltpu.SemaphoreType.DMA,
            pltpu.SemaphoreType.DMA,
        ],
    )


def kernel(z, embedding):
    z3 = z.reshape(B, C, HW)
    idx3, stats = _tc_call(embedding, z3)
    idx_flat = idx3.reshape(N)
    zq_flat = _make_sc_gather()(embedding, idx_flat)
    z_q = zq_flat.reshape(B, HW, C).transpose(0, 2, 1).reshape(B, C, H, W)
    loss = jnp.zeros((), jnp.float32)
    perplexity = stats[0]
    mean_distance = stats[1]
    indices = idx3.reshape(B, H, W)
    return (z_q, loss, perplexity, indices, mean_distance)


# speedup vs baseline: 1.1070x; 1.1070x over previous
import functools

import jax
import jax.numpy as jnp
from jax import lax
from jax.experimental import pallas as pl
from jax.experimental.pallas import tpu as pltpu
from jax.experimental.pallas import tpu_sc as plsc

B, C, H, W = 16, 256, 32, 32
HW = H * W
N = B * HW
K = 1024
D = C

NC = 2
NS = 16
NW = NC * NS
ROWS_PER_W = N // NW
CHUNK = 128
NCHUNK = ROWS_PER_W // CHUNK


def _tc_body(emb_ref, z_ref, idx_ref, stats_ref, counts_ref, acc_ref):
    b = pl.program_id(0)
    emb = emb_ref[...]
    zb = z_ref[0]
    s_t = jax.lax.dot_general(
        emb, zb, (((1,), (0,)), ((), ())),
        preferred_element_type=jnp.float32,
        precision=lax.Precision.DEFAULT,
    )
    enorm = jnp.sum(emb * emb, axis=1, keepdims=True)
    znorm = jnp.sum(zb * zb, axis=0, keepdims=True)
    dist_t = (znorm - 2.0 * s_t) + enorm
    m = jnp.min(dist_t, axis=0, keepdims=True)
    ks = lax.broadcasted_iota(jnp.int32, (K, HW), 0)
    eq = dist_t == m
    idx = jnp.min(jnp.where(eq, ks, K), axis=0).astype(jnp.int32)
    idx_ref[0, 0, :] = idx

    @pl.when(b == 0)
    def _init():
        counts_ref[...] = jnp.zeros_like(counts_ref)
        acc_ref[0] = 0.0

    ones = jnp.ones((HW, 1), jnp.float32)
    counts_ref[...] += jax.lax.dot_general(
        eq.astype(jnp.float32), ones, (((1,), (0,)), ((), ())),
        preferred_element_type=jnp.float32)
    esum = jnp.sum(emb, axis=0, keepdims=True)
    zsum = jnp.sum(zb, axis=1, keepdims=True)
    cross = jax.lax.dot_general(
        esum, zsum, (((1,), (0,)), ((), ())),
        preferred_element_type=jnp.float32,
        precision=lax.Precision.HIGHEST)
    acc_ref[0] += (K * jnp.sum(znorm) + HW * jnp.sum(enorm)
                   - 2.0 * cross[0, 0])

    @pl.when(b == B - 1)
    def _finalize():
        e_mean = counts_ref[...] * (1.0 / N)
        ent = jnp.sum(e_mean * jnp.log(e_mean + 1e-10))
        stats_ref[0] = jnp.exp(-ent)
        stats_ref[1] = acc_ref[0] * (1.0 / (N * K))


_tc_call = pl.pallas_call(
    _tc_body,
    grid=(B,),
    in_specs=[
        pl.BlockSpec((K, D), lambda b: (0, 0)),
        pl.BlockSpec((1, C, HW), lambda b: (b, 0, 0)),
    ],
    out_specs=[
        pl.BlockSpec((1, 1, HW), lambda b: (b, 0, 0)),
        pl.BlockSpec(memory_space=pltpu.SMEM),
    ],
    out_shape=[
        jax.ShapeDtypeStruct((B, 1, HW), jnp.int32),
        jax.ShapeDtypeStruct((2,), jnp.float32),
    ],
    scratch_shapes=[
        pltpu.VMEM((K, 1), jnp.float32),
        pltpu.SMEM((2,), jnp.float32),
    ],
)


def _sc_gather_body(emb_hbm, idx_hbm, out_hbm, idx_v, buf0, buf1, sem0, sem1):
    c = lax.axis_index("c")
    s = lax.axis_index("s")
    wid = s * NC + c
    base = wid * ROWS_PER_W
    pltpu.sync_copy(idx_hbm.at[pl.ds(base, ROWS_PER_W)], idx_v)
    bufs = (buf0, buf1)
    sems = (sem0, sem1)
    copies = [None, None]
    copies[0] = pltpu.async_copy(
        emb_hbm.at[idx_v.at[pl.ds(0, CHUNK)]], bufs[0], sems[0])
    for j in range(NCHUNK):
        cur = j % 2
        if j + 1 < NCHUNK:
            nxt = (j + 1) % 2
            copies[nxt] = pltpu.async_copy(
                emb_hbm.at[idx_v.at[pl.ds((j + 1) * CHUNK, CHUNK)]],
                bufs[nxt], sems[nxt])
        copies[cur].wait()
        pltpu.sync_copy(bufs[cur], out_hbm.at[pl.ds(base + j * CHUNK, CHUNK)])


@functools.lru_cache(maxsize=1)
def _make_sc_gather():
    return pl.kernel(
        _sc_gather_body,
        out_type=jax.ShapeDtypeStruct((N, D), jnp.float32),
        mesh=plsc.VectorSubcoreMesh(
            core_axis_name="c", subcore_axis_name="s",
            num_cores=NC, num_subcores=NS),
        scratch_types=[
            pltpu.VMEM((ROWS_PER_W,), jnp.int32),
            pltpu.VMEM((CHUNK, D), jnp.float32),
            pltpu.VMEM((CHUNK, D), jnp.float32),
            pltpu.SemaphoreType.DMA,
            pltpu.SemaphoreType.DMA,
        ],
    )


def kernel(z, embedding):
    z3 = z.reshape(B, C, HW)
    idx3, stats = _tc_call(embedding, z3)
    idx_flat = idx3.reshape(N)
    zq_flat = _make_sc_gather()(embedding, idx_flat)
    z_q = zq_flat.reshape(B, HW, C).transpose(0, 2, 1).reshape(B, C, H, W)
    loss = jnp.zeros((), jnp.float32)
    perplexity = stats[0]
    mean_distance = stats[1]
    indices = idx3.reshape(B, H, W)
    return (z_q, loss, perplexity, indices, mean_distance)
